# baseline (speedup 1.0000x reference)
import functools

import jax
import jax.numpy as jnp
from jax import lax
from jax.experimental import pallas as pl
from jax.experimental.pallas import tpu as pltpu
from jax.experimental.pallas import tpu_sc as plsc

N = 10000
E = 320000
D = 128
MAX_Z = 100
NUM_BASIS = 16
H = 64
NG = 256

NC = 2
NS = 16
NW = NC * NS
EPW = E // NW
C = 40
NCH = EPW // C
RPS = 624
NTAIL = N - NS * RPS

_mesh = lambda: plsc.VectorSubcoreMesh(core_axis_name="c", subcore_axis_name="s")


CE = 80


def _emb_body(emb_hbm, z_hbm, out_hbm, idx_v, rows_v, sem):
    cid = lax.axis_index("c")
    sid = lax.axis_index("s")
    wid = sid * NC + cid
    for t in range(4):
        r = wid * 4 + t

        @pl.when(r < N // CE)
        def _(r=r):
            pltpu.sync_copy(z_hbm.at[r], idx_v)
            pltpu.async_copy(emb_hbm.at[idx_v], rows_v, sem).wait()
            pltpu.sync_copy(rows_v, out_hbm.at[pl.ds(r * CE, CE)])


def _embed(emb_table, z2):
    f = functools.partial(
        pl.kernel,
        out_type=jax.ShapeDtypeStruct((N, D), jnp.float32),
        mesh=_mesh(),
        scratch_types=[
            pltpu.VMEM((CE,), jnp.int32),
            pltpu.VMEM((CE, D), jnp.float32),
            pltpu.SemaphoreType.DMA,
        ],
    )(_emb_body)
    return f(emb_table, z2)


BE = 2560


def _radial_body(d_ref, n_ref, w1_ref, b1c_ref, w2a_ref, out_ref):
    d = d_ref[0]
    nrm = n_ref[0]
    centers = lax.broadcasted_iota(jnp.int32, (NUM_BASIS, 1), 0).astype(
        jnp.float32) * (1.0 / (NUM_BASIS - 1))
    diff = d - centers
    inv2w2 = 0.5 * float(NUM_BASIS) * float(NUM_BASIS)
    rbf_t = jnp.exp(-(diff * diff) * inv2w2)
    dn = (((0,), (0,)), ((), ()))
    h_t = lax.dot_general(w1_ref[...], rbf_t.astype(jnp.bfloat16), dn,
                          preferred_element_type=jnp.float32)
    h_t = h_t + b1c_ref[...]
    h_t = h_t * jax.nn.sigmoid(h_t)
    h_aug = jnp.concatenate([h_t * nrm, nrm], axis=0)
    rwp = lax.dot_general(h_aug.astype(jnp.bfloat16), w2a_ref[...],
                          dn, preferred_element_type=jnp.float32)

    def rne16(x):
        xi = lax.bitcast_convert_type(x, jnp.int32)
        return xi + jnp.int32(0x7FFF) + ((xi >> 16) & 1)

    lo = lax.shift_right_logical(rne16(rwp[:, : D // 2]), 16)
    hi = rne16(rwp[:, D // 2:]) & jnp.int32(-65536)
    out_ref[...] = lo | hi


def _radial(d2, n2, W1, b1c, W2a):
    grid = E // BE
    return pl.pallas_call(
        _radial_body,
        grid=(grid,),
        in_specs=[
            pl.BlockSpec((1, 1, BE), lambda i: (i, 0, 0)),
            pl.BlockSpec((1, 1, BE), lambda i: (i, 0, 0)),
            pl.BlockSpec((NUM_BASIS, H), lambda i: (0, 0)),
            pl.BlockSpec((H, 1), lambda i: (0, 0)),
            pl.BlockSpec((H + 1, D), lambda i: (0, 0)),
        ],
        out_specs=pl.BlockSpec((BE, D // 2), lambda i: (i, 0)),
        out_shape=jax.ShapeDtypeStruct((E, D // 2), jnp.int32),
    )(d2, n2, W1, b1c, W2a)


def _msg_body(feat_hbm, rw_hbm, sd_hbm, out_hbm,
              sd, rw, g, zero_v, agg_sh, isem, rsem, gsem, ssem):
    cid = lax.axis_index("c")
    sid = lax.axis_index("s")
    wid = sid * NC + cid

    def issue_idx(c, s):
        pltpu.async_copy(sd_hbm.at[wid, c], sd[s], isem[s])

    def wait_idx(s):
        pltpu.make_async_copy(sd_hbm.at[wid, 0], sd[s], isem[s]).wait()

    def issue_gather(s):
        pltpu.async_copy(feat_hbm.at[sd[s].at[0]], g[s], gsem[s])

    def wait_gather(s):
        pltpu.make_async_copy(feat_hbm.at[sd[s].at[0]], g[s], gsem[s]).wait()

    def issue_rw(c, s):
        pltpu.async_copy(rw_hbm.at[pl.ds(wid * EPW + c * C, C)], rw[s],
                         rsem[s])

    def wait_rw(s):
        pltpu.make_async_copy(rw_hbm.at[pl.ds(0, C)], rw[s], rsem[s]).wait()

    def compute(s):
        def mul(i, _):
            for j in range(D // 32):
                x = rw[s][i, pl.ds(16 * j, 16)]
                a = lax.bitcast_convert_type(lax.shift_left(x, 16),
                                             jnp.float32)
                b = lax.bitcast_convert_type(x & jnp.int32(-65536),
                                             jnp.float32)
                sa = pl.ds(32 * j, 16)
                sb = pl.ds(32 * j + 16, 16)
                g[s][i, sa] = g[s][i, sa] * a
                g[s][i, sb] = g[s][i, sb] * b
            return 0
        lax.fori_loop(0, C, mul, 0)

    def issue_scatter(s):
        pltpu.async_copy(g[s], agg_sh.at[sd[s].at[1]], ssem, add=True)

    def wait_scatter(s):
        pltpu.make_async_copy(g[s], agg_sh.at[sd[s].at[1]], ssem).wait()

    for s in range(3):
        issue_idx(s, s)
    wait_idx(0)
    issue_gather(0)
    issue_rw(0, 0)
    wait_idx(1)
    issue_gather(1)
    issue_rw(1, 1)

    for i in range(16):
        for j in range(D // 16):
            zero_v[i, pl.ds(j * 16, 16)] = jnp.zeros((16,), jnp.float32)
    for t in range(RPS // 16):
        pltpu.sync_copy(zero_v, agg_sh.at[pl.ds(sid * RPS + t * 16, 16)])

    @pl.when(sid == NS - 1)
    def _():
        pltpu.sync_copy(zero_v, agg_sh.at[pl.ds(NS * RPS, NTAIL)])
    plsc.subcore_barrier()

    def step(c, s, tail):
        wait_gather(s)
        wait_rw(s)
        compute(s)
        if tail:
            wait_scatter((s + 3) % 4)
        else:
            @pl.when(c > 0)
            def _():
                wait_scatter((s + 3) % 4)
        issue_scatter(s)
        if not tail:
            @pl.when(c + 3 < NCH)
            def _():
                issue_idx(c + 3, (s + 3) % 4)

            @pl.when(c + 2 < NCH)
            def _():
                wait_idx((s + 2) % 4)
                issue_gather((s + 2) % 4)
                issue_rw(c + 2, (s + 2) % 4)

    def quad(q, _):
        for r in range(4):
            step(4 * q + r, r, False)
        return 0
    lax.fori_loop(0, NCH // 4, quad, 0)
    for c in range(NCH - NCH % 4, NCH):
        step(c, c % 4, True)
    wait_scatter((NCH - 1) % 4)
    plsc.subcore_barrier()

    pltpu.sync_copy(agg_sh.at[pl.ds(sid * RPS, RPS)],
                    out_hbm.at[cid, pl.ds(sid * RPS, RPS)])

    @pl.when(sid == NS - 1)
    def _():
        pltpu.sync_copy(agg_sh.at[pl.ds(NS * RPS, NTAIL)],
                        out_hbm.at[cid, pl.ds(NS * RPS, NTAIL)])


def _messages(feats, rw, sd4):
    f = functools.partial(
        pl.kernel,
        out_type=jax.ShapeDtypeStruct((NC, N, D), jnp.float32),
        mesh=_mesh(),
        scratch_types=[
            [pltpu.VMEM((2, C), jnp.int32) for _ in range(4)],
            [pltpu.VMEM((C, D // 2), jnp.int32) for _ in range(4)],
            [pltpu.VMEM((C, D), jnp.float32) for _ in range(4)],
            pltpu.VMEM((16, D), jnp.float32),
            pltpu.VMEM_SHARED((N, D), jnp.float32),
            [pltpu.SemaphoreType.DMA for _ in range(4)],
            [pltpu.SemaphoreType.DMA for _ in range(4)],
            [pltpu.SemaphoreType.DMA for _ in range(4)],
            pltpu.SemaphoreType.DMA,
        ],
    )(_msg_body)
    return f(feats, rw, sd4)


BN = 2000
NBB = N // BN


def _head_body(part_ref, batch_ref, w1_ref, b1_ref, w2_ref, out_ref,
               sums, counts):
    pid = pl.program_id(0)

    @pl.when(pid == 0)
    def _():
        sums[...] = jnp.zeros_like(sums)
        counts[...] = jnp.zeros_like(counts)

    a = part_ref[0] + part_ref[1]
    g = a * jax.nn.sigmoid(a)
    b = batch_ref[0]
    gids = lax.broadcasted_iota(jnp.int32, (NG, 1), 0)
    oh = (b == gids).astype(jnp.float32)
    sums[...] += jnp.dot(oh, g, preferred_element_type=jnp.float32)
    counts[...] += jnp.sum(oh, axis=1, keepdims=True)

    @pl.when(pid == NBB - 1)
    def _():
        pooled = sums[...] / jnp.maximum(counts[...], 1.0)
        hfc = pooled @ w1_ref[...] + b1_ref[...]
        hfc = jnp.maximum(hfc, 0.0)
        out_ref[...] = jnp.dot(hfc, w2_ref[...],
                               preferred_element_type=jnp.float32)


def _head(partials, batch3, fcW1, fcb1r, fcW2):
    return pl.pallas_call(
        _head_body,
        grid=(NBB,),
        in_specs=[
            pl.BlockSpec((NC, BN, D), lambda i: (0, i, 0)),
            pl.BlockSpec((1, 1, BN), lambda i: (i, 0, 0)),
            pl.BlockSpec((D, D), lambda i: (0, 0)),
            pl.BlockSpec((1, D), lambda i: (0, 0)),
            pl.BlockSpec((D, 1), lambda i: (0, 0)),
        ],
        out_specs=pl.BlockSpec((NG, 1), lambda i: (0, 0)),
        out_shape=jax.ShapeDtypeStruct((NG, 1), jnp.float32),
        scratch_shapes=[
            pltpu.VMEM((NG, D), jnp.float32),
            pltpu.VMEM((NG, 1), jnp.float32),
        ],
    )(partials, batch3, fcW1, fcb1r, fcW2)


def kernel(z, edge_index, abs_distances, rel_vec, norm, batch,
           emb_table, W1, b1, W2, b2, fcW1, fcb1, fcW2, fcb2):
    del rel_vec
    feats = _embed(emb_table, z.astype(jnp.int32).reshape(N // CE, CE))
    w2a = jnp.concatenate([W2, b2[None, :]], axis=0)
    perm = ([32 * (k // 16) + k % 16 for k in range(D // 2)]
            + [32 * (k // 16) + 16 + k % 16 for k in range(D // 2)])
    w2a_p = w2a[:, jnp.array(perm, dtype=jnp.int32)]
    rw = _radial(abs_distances.reshape(E // BE, 1, BE),
                 norm.reshape(E // BE, 1, BE),
                 W1.astype(jnp.bfloat16), b1.reshape(H, 1),
                 w2a_p.astype(jnp.bfloat16))
    ei = edge_index.astype(jnp.int32)
    sd4 = jnp.stack([ei[0].reshape(NW, NCH, C), ei[1].reshape(NW, NCH, C)],
                    axis=2)
    partials = _messages(feats, rw, sd4)
    out = _head(partials, batch.astype(jnp.int32).reshape(NBB, 1, BN),
                fcW1, fcb1.reshape(1, D), fcW2)
    return out + fcb2[None, :]


# speedup vs baseline: 1.0096x; 1.0096x over previous
import functools

import jax
import jax.numpy as jnp
from jax import lax
from jax.experimental import pallas as pl
from jax.experimental.pallas import tpu as pltpu
from jax.experimental.pallas import tpu_sc as plsc

N = 10000
E = 320000
D = 128
MAX_Z = 100
NUM_BASIS = 16
H = 64
NG = 256

NC = 2
NS = 16
NW = NC * NS
EPW = E // NW
C = 40
NCH = EPW // C
RPS = 624
NTAIL = N - NS * RPS

_mesh = lambda: plsc.VectorSubcoreMesh(core_axis_name="c", subcore_axis_name="s")


CE = 80


def _emb_body(emb_hbm, z_hbm, out_hbm, idx_v, rows_v, sem):
    cid = lax.axis_index("c")
    sid = lax.axis_index("s")
    wid = sid * NC + cid
    for t in range(4):
        r = wid * 4 + t

        @pl.when(r < N // CE)
        def _(r=r):
            pltpu.sync_copy(z_hbm.at[r], idx_v)
            pltpu.async_copy(emb_hbm.at[idx_v], rows_v, sem).wait()
            pltpu.sync_copy(rows_v, out_hbm.at[pl.ds(r * CE, CE)])


def _embed(emb_table, z2):
    f = functools.partial(
        pl.kernel,
        out_type=jax.ShapeDtypeStruct((N, D), jnp.float32),
        mesh=_mesh(),
        scratch_types=[
            pltpu.VMEM((CE,), jnp.int32),
            pltpu.VMEM((CE, D), jnp.float32),
            pltpu.SemaphoreType.DMA,
        ],
    )(_emb_body)
    return f(emb_table, z2)


BE = 2560


def _radial_body(d_ref, n_ref, w1_ref, b1c_ref, w2a_ref, out_ref):
    d = d_ref[0]
    nrm = n_ref[0]
    centers = lax.broadcasted_iota(jnp.int32, (NUM_BASIS, 1), 0).astype(
        jnp.float32) * (1.0 / (NUM_BASIS - 1))
    diff = d - centers
    inv2w2 = 0.5 * float(NUM_BASIS) * float(NUM_BASIS)
    rbf_t = jnp.exp(-(diff * diff) * inv2w2)
    dn = (((0,), (0,)), ((), ()))
    h_t = lax.dot_general(w1_ref[...], rbf_t.astype(jnp.bfloat16), dn,
                          preferred_element_type=jnp.float32)
    h_t = h_t + b1c_ref[...]
    h_t = h_t * jax.nn.sigmoid(h_t)
    h_aug = jnp.concatenate([h_t * nrm, nrm], axis=0)
    rwp = lax.dot_general(h_aug.astype(jnp.bfloat16), w2a_ref[...],
                          dn, preferred_element_type=jnp.float32)

    def r16(x):
        return lax.bitcast_convert_type(x, jnp.int32) + jnp.int32(0x8000)

    lo = lax.shift_right_logical(r16(rwp[:, : D // 2]), 16)
    hi = r16(rwp[:, D // 2:]) & jnp.int32(-65536)
    out_ref[...] = lo | hi


def _radial(d2, n2, W1, b1c, W2a):
    grid = E // BE
    return pl.pallas_call(
        _radial_body,
        grid=(grid,),
        in_specs=[
            pl.BlockSpec((1, 1, BE), lambda i: (i, 0, 0)),
            pl.BlockSpec((1, 1, BE), lambda i: (i, 0, 0)),
            pl.BlockSpec((NUM_BASIS, H), lambda i: (0, 0)),
            pl.BlockSpec((H, 1), lambda i: (0, 0)),
            pl.BlockSpec((H + 1, D), lambda i: (0, 0)),
        ],
        out_specs=pl.BlockSpec((BE, D // 2), lambda i: (i, 0)),
        out_shape=jax.ShapeDtypeStruct((E, D // 2), jnp.int32),
    )(d2, n2, W1, b1c, W2a)


def _msg_body(feat_hbm, rw_hbm, sd_hbm, out_hbm,
              sd, rw, g, zero_v, agg_sh, isem, rsem, gsem, ssem):
    cid = lax.axis_index("c")
    sid = lax.axis_index("s")
    wid = sid * NC + cid

    def issue_idx(c, s):
        pltpu.async_copy(sd_hbm.at[wid, c], sd[s], isem[s])

    def wait_idx(s):
        pltpu.make_async_copy(sd_hbm.at[wid, 0], sd[s], isem[s]).wait()

    def issue_gather(s):
        pltpu.async_copy(feat_hbm.at[sd[s].at[0]], g[s], gsem[s])

    def wait_gather(s):
        pltpu.make_async_copy(feat_hbm.at[sd[s].at[0]], g[s], gsem[s]).wait()

    def issue_rw(c, s):
        pltpu.async_copy(rw_hbm.at[pl.ds(wid * EPW + c * C, C)], rw[s],
                         rsem[s])

    def wait_rw(s):
        pltpu.make_async_copy(rw_hbm.at[pl.ds(0, C)], rw[s], rsem[s]).wait()

    def compute(s):
        def mul(i, _):
            for j in range(D // 32):
                x = rw[s][i, pl.ds(16 * j, 16)]
                a = lax.bitcast_convert_type(lax.shift_left(x, 16),
                                             jnp.float32)
                b = lax.bitcast_convert_type(x & jnp.int32(-65536),
                                             jnp.float32)
                sa = pl.ds(32 * j, 16)
                sb = pl.ds(32 * j + 16, 16)
                g[s][i, sa] = g[s][i, sa] * a
                g[s][i, sb] = g[s][i, sb] * b
            return 0
        lax.fori_loop(0, C, mul, 0)

    def issue_scatter(s):
        pltpu.async_copy(g[s], agg_sh.at[sd[s].at[1]], ssem, add=True)

    def wait_scatter(s):
        pltpu.make_async_copy(g[s], agg_sh.at[sd[s].at[1]], ssem).wait()

    for s in range(3):
        issue_idx(s, s)
    wait_idx(0)
    issue_gather(0)
    issue_rw(0, 0)
    wait_idx(1)
    issue_gather(1)
    issue_rw(1, 1)

    for i in range(16):
        for j in range(D // 16):
            zero_v[i, pl.ds(j * 16, 16)] = jnp.zeros((16,), jnp.float32)
    for t in range(RPS // 16):
        pltpu.sync_copy(zero_v, agg_sh.at[pl.ds(sid * RPS + t * 16, 16)])

    @pl.when(sid == NS - 1)
    def _():
        pltpu.sync_copy(zero_v, agg_sh.at[pl.ds(NS * RPS, NTAIL)])
    plsc.subcore_barrier()

    def step(c, s, tail):
        wait_gather(s)
        wait_rw(s)
        compute(s)
        if tail:
            wait_scatter((s + 3) % 4)
        else:
            @pl.when(c > 0)
            def _():
                wait_scatter((s + 3) % 4)
        issue_scatter(s)
        if not tail:
            @pl.when(c + 3 < NCH)
            def _():
                issue_idx(c + 3, (s + 3) % 4)

            @pl.when(c + 2 < NCH)
            def _():
                wait_idx((s + 2) % 4)
                issue_gather((s + 2) % 4)
                issue_rw(c + 2, (s + 2) % 4)

    def quad(q, _):
        for r in range(4):
            step(4 * q + r, r, False)
        return 0
    lax.fori_loop(0, NCH // 4, quad, 0)
    for c in range(NCH - NCH % 4, NCH):
        step(c, c % 4, True)
    wait_scatter((NCH - 1) % 4)
    plsc.subcore_barrier()

    pltpu.sync_copy(agg_sh.at[pl.ds(sid * RPS, RPS)],
                    out_hbm.at[cid, pl.ds(sid * RPS, RPS)])

    @pl.when(sid == NS - 1)
    def _():
        pltpu.sync_copy(agg_sh.at[pl.ds(NS * RPS, NTAIL)],
                        out_hbm.at[cid, pl.ds(NS * RPS, NTAIL)])


def _messages(feats, rw, sd4):
    f = functools.partial(
        pl.kernel,
        out_type=jax.ShapeDtypeStruct((NC, N, D), jnp.float32),
        mesh=_mesh(),
        scratch_types=[
            [pltpu.VMEM((2, C), jnp.int32) for _ in range(4)],
            [pltpu.VMEM((C, D // 2), jnp.int32) for _ in range(4)],
            [pltpu.VMEM((C, D), jnp.float32) for _ in range(4)],
            pltpu.VMEM((16, D), jnp.float32),
            pltpu.VMEM_SHARED((N, D), jnp.float32),
            [pltpu.SemaphoreType.DMA for _ in range(4)],
            [pltpu.SemaphoreType.DMA for _ in range(4)],
            [pltpu.SemaphoreType.DMA for _ in range(4)],
            pltpu.SemaphoreType.DMA,
        ],
    )(_msg_body)
    return f(feats, rw, sd4)


BN = 2000
NBB = N // BN


def _head_body(part_ref, batch_ref, w1_ref, b1_ref, w2_ref, out_ref,
               sums, counts):
    pid = pl.program_id(0)

    @pl.when(pid == 0)
    def _():
        sums[...] = jnp.zeros_like(sums)
        counts[...] = jnp.zeros_like(counts)

    a = part_ref[0] + part_ref[1]
    g = a * jax.nn.sigmoid(a)
    b = batch_ref[0]
    gids = lax.broadcasted_iota(jnp.int32, (NG, 1), 0)
    oh = (b == gids).astype(jnp.float32)
    sums[...] += jnp.dot(oh, g, preferred_element_type=jnp.float32)
    counts[...] += jnp.sum(oh, axis=1, keepdims=True)

    @pl.when(pid == NBB - 1)
    def _():
        pooled = sums[...] / jnp.maximum(counts[...], 1.0)
        hfc = pooled @ w1_ref[...] + b1_ref[...]
        hfc = jnp.maximum(hfc, 0.0)
        out_ref[...] = jnp.dot(hfc, w2_ref[...],
                               preferred_element_type=jnp.float32)


def _head(partials, batch3, fcW1, fcb1r, fcW2):
    return pl.pallas_call(
        _head_body,
        grid=(NBB,),
        in_specs=[
            pl.BlockSpec((NC, BN, D), lambda i: (0, i, 0)),
            pl.BlockSpec((1, 1, BN), lambda i: (i, 0, 0)),
            pl.BlockSpec((D, D), lambda i: (0, 0)),
            pl.BlockSpec((1, D), lambda i: (0, 0)),
            pl.BlockSpec((D, 1), lambda i: (0, 0)),
        ],
        out_specs=pl.BlockSpec((NG, 1), lambda i: (0, 0)),
        out_shape=jax.ShapeDtypeStruct((NG, 1), jnp.float32),
        scratch_shapes=[
            pltpu.VMEM((NG, D), jnp.float32),
            pltpu.VMEM((NG, 1), jnp.float32),
        ],
    )(partials, batch3, fcW1, fcb1r, fcW2)


def kernel(z, edge_index, abs_distances, rel_vec, norm, batch,
           emb_table, W1, b1, W2, b2, fcW1, fcb1, fcW2, fcb2):
    del rel_vec
    feats = _embed(emb_table, z.astype(jnp.int32).reshape(N // CE, CE))
    w2a = jnp.concatenate([W2, b2[None, :]], axis=0)
    perm = ([32 * (k // 16) + k % 16 for k in range(D // 2)]
            + [32 * (k // 16) + 16 + k % 16 for k in range(D // 2)])
    w2a_p = w2a[:, jnp.array(perm, dtype=jnp.int32)]
    rw = _radial(abs_distances.reshape(E // BE, 1, BE),
                 norm.reshape(E // BE, 1, BE),
                 W1.astype(jnp.bfloat16), b1.reshape(H, 1),
                 w2a_p.astype(jnp.bfloat16))
    ei = edge_index.astype(jnp.int32)
    sd4 = jnp.stack([ei[0].reshape(NW, NCH, C), ei[1].reshape(NW, NCH, C)],
                    axis=2)
    partials = _messages(feats, rw, sd4)
    out = _head(partials, batch.astype(jnp.int32).reshape(NBB, 1, BN),
                fcW1, fcb1.reshape(1, D), fcW2)
    return out + fcb2[None, :]
